# Initial kernel scaffold; baseline (speedup 1.0000x reference)
import jax
import jax.numpy as jnp
from jax.experimental import pallas as pl


def kernel(net_mask, depth_hat, depth, k):
    raise NotImplementedError("write your pallas kernel here")


# speedup vs baseline: 53.1110x; 53.1110x over previous
import functools

import jax
import jax.numpy as jnp
from jax import lax
from jax.experimental import pallas as pl
from jax.experimental.pallas import tpu as pltpu
from jax.experimental.pallas import tpu_sc as plsc

N_TOTAL = 2097152
K_MASK = 65536
NC = 2
NS = 16
LANES = 16
NW = NC * NS
PER_W = N_TOTAL // NW
CHUNK = 16384
N_CHUNKS = PER_W // CHUNK
UNROLL = 8


def _sc_body(dh_hbm, d_hbm, out_hbm, dh_v, d_v, head_a, head_b, row_v):
    wid = lax.axis_index("s") * NC + lax.axis_index("c")
    base = wid * PER_W

    pltpu.sync_copy(dh_hbm.at[pl.ds(0, LANES)], head_a)
    pltpu.sync_copy(d_hbm.at[pl.ds(0, LANES)], head_b)
    diff0 = jnp.abs(head_a[...] - head_b[...])
    g_head = jnp.exp(-(diff0 * diff0))
    head_a[...] = g_head
    g0v = plsc.load_gather(head_a, [jnp.zeros((LANES,), jnp.int32)])

    def chunk_step(c, acc):
        off = base + c * CHUNK
        pltpu.sync_copy(dh_hbm.at[pl.ds(off, CHUNK)], dh_v)
        pltpu.sync_copy(d_hbm.at[pl.ds(off, CHUNK)], d_v)

        def inner(i, a):
            s0 = i * (LANES * UNROLL)
            for u in range(UNROLL):
                s = s0 + u * LANES
                x = dh_v[pl.ds(s, LANES)] - d_v[pl.ds(s, LANES)]
                ax = jnp.abs(x)
                g = jnp.exp(-(ax * ax))
                a = a + jnp.where(g > g0v, 1, 0).astype(jnp.int32)
            return a

        return lax.fori_loop(0, CHUNK // (LANES * UNROLL), inner, acc)

    acc = lax.fori_loop(0, N_CHUNKS, chunk_step,
                        jnp.zeros((LANES,), jnp.int32))
    row_v[...] = acc
    pltpu.sync_copy(row_v, out_hbm.at[wid])


_sc_count = pl.kernel(
    _sc_body,
    out_type=jax.ShapeDtypeStruct((NW, LANES), jnp.int32),
    mesh=plsc.VectorSubcoreMesh(
        core_axis_name="c", subcore_axis_name="s",
        num_cores=NC, num_subcores=NS),
    compiler_params=pltpu.CompilerParams(needs_layout_passes=False),
    scratch_types=[
        pltpu.VMEM((CHUNK,), jnp.float32),
        pltpu.VMEM((CHUNK,), jnp.float32),
        pltpu.VMEM((LANES,), jnp.float32),
        pltpu.VMEM((LANES,), jnp.float32),
        pltpu.VMEM((LANES,), jnp.int32),
    ],
)


def _tc_body(part_ref, nm_ref, out_ref):
    p = jnp.sum(part_ref[...])
    nm = nm_ref[...]
    nz = (nm != 0.0).astype(jnp.int32)
    a_cnt = jnp.sum(nz)
    rows = lax.broadcasted_iota(jnp.int32, nm.shape, 0)
    cols = lax.broadcasted_iota(jnp.int32, nm.shape, 1)
    lin = rows * nm.shape[1] + cols
    hit = jnp.sum(jnp.where((lin == p) & (nz == 1), 1, 0))
    in_topk = p < K_MASK
    inter = jnp.where(in_topk, a_cnt - hit, a_cnt)
    union = jnp.where(in_topk, (K_MASK - 1) + hit, K_MASK)
    iou = inter.astype(jnp.float32) / union.astype(jnp.float32)
    val = jnp.log(jnp.full((8, 128), 1.0 / iou, jnp.float32))
    out_ref[0, 0] = val[0, 0]


_tc_final = pl.pallas_call(
    _tc_body,
    out_shape=jax.ShapeDtypeStruct((1, 1), jnp.float32),
    in_specs=[
        pl.BlockSpec(memory_space=pltpu.VMEM),
        pl.BlockSpec(memory_space=pltpu.VMEM),
    ],
    out_specs=pl.BlockSpec(memory_space=pltpu.SMEM),
)


def kernel(net_mask, depth_hat, depth, k):
    partials = _sc_count(depth_hat, depth)
    part4 = partials.reshape(4, 128)
    nm2 = net_mask.reshape(K_MASK // 1024, 1024)
    out = _tc_final(part4, nm2)
    return out[0, 0]


# speedup vs baseline: 65.3692x; 1.2308x over previous
import jax
import jax.numpy as jnp
from jax import lax
from jax.experimental import pallas as pl
from jax.experimental.pallas import tpu as pltpu
from jax.experimental.pallas import tpu_sc as plsc

N_TOTAL = 2097152
K_MASK = 65536
NC = 2
NS = 16
LANES = 16
NW = NC * NS
PER_W = N_TOTAL // NW
CHUNK = 16384
N_CHUNKS = PER_W // CHUNK
UNROLL = 16
INF_BITS = 0x7F800000


def _sc_body(dh_hbm, d_hbm, out_hbm,
             dh_a, d_a, dh_b, d_b, head_a, head_b, row_v,
             sem_dh_a, sem_d_a, sem_dh_b, sem_d_b):
    wid = lax.axis_index("s") * NC + lax.axis_index("c")
    base = wid * PER_W

    pltpu.sync_copy(dh_hbm.at[pl.ds(0, LANES)], head_a)
    pltpu.sync_copy(d_hbm.at[pl.ds(0, LANES)], head_b)
    diff0 = jnp.abs(head_a[...] - head_b[...])
    g_head = jnp.exp(-(diff0 * diff0))
    head_a[...] = g_head
    g0v = plsc.load_gather(head_a, [jnp.zeros((LANES,), jnp.int32)])

    def bis(_, lh):
        lo, hi = lh
        mid = lo + lax.div(hi - lo, 2)
        t = plsc.bitcast(mid, jnp.float32)
        pfalse = jnp.exp(-t) <= g0v
        return (jnp.where(pfalse, lo, mid + 1),
                jnp.where(pfalse, mid, hi))

    _, hi = lax.fori_loop(
        0, 32, bis,
        (jnp.zeros((LANES,), jnp.int32),
         jnp.full((LANES,), INF_BITS, jnp.int32)))
    tstar = plsc.bitcast(hi, jnp.float32)

    dh_bufs, d_bufs = (dh_a, dh_b), (d_a, d_b)
    dh_sems, d_sems = (sem_dh_a, sem_dh_b), (sem_d_a, sem_d_b)

    def start(c, b):
        off = base + c * CHUNK
        return (pltpu.async_copy(dh_hbm.at[pl.ds(off, CHUNK)],
                                 dh_bufs[b], dh_sems[b]),
                pltpu.async_copy(d_hbm.at[pl.ds(off, CHUNK)],
                                 d_bufs[b], d_sems[b]))

    acc = jnp.zeros((LANES,), jnp.int32)
    pend = start(0, 0)
    for c in range(N_CHUNKS):
        nxt = start(c + 1, (c + 1) % 2) if c + 1 < N_CHUNKS else None
        for h in pend:
            h.wait()
        dh_v, d_v = dh_bufs[c % 2], d_bufs[c % 2]

        def inner(i, a, dh_v=dh_v, d_v=d_v):
            s0 = i * (LANES * UNROLL)
            for u in range(UNROLL):
                s = s0 + u * LANES
                x = dh_v[pl.ds(s, LANES)] - d_v[pl.ds(s, LANES)]
                a = a + jnp.where(x * x < tstar, 1, 0).astype(jnp.int32)
            return a

        acc = lax.fori_loop(0, CHUNK // (LANES * UNROLL), inner, acc)
        pend = nxt
    row_v[...] = acc
    pltpu.sync_copy(row_v, out_hbm.at[wid])


_sc_count = pl.kernel(
    _sc_body,
    out_type=jax.ShapeDtypeStruct((NW, LANES), jnp.int32),
    mesh=plsc.VectorSubcoreMesh(
        core_axis_name="c", subcore_axis_name="s",
        num_cores=NC, num_subcores=NS),
    compiler_params=pltpu.CompilerParams(needs_layout_passes=False),
    scratch_types=[
        pltpu.VMEM((CHUNK,), jnp.float32),
        pltpu.VMEM((CHUNK,), jnp.float32),
        pltpu.VMEM((CHUNK,), jnp.float32),
        pltpu.VMEM((CHUNK,), jnp.float32),
        pltpu.VMEM((LANES,), jnp.float32),
        pltpu.VMEM((LANES,), jnp.float32),
        pltpu.VMEM((LANES,), jnp.int32),
        pltpu.SemaphoreType.DMA,
        pltpu.SemaphoreType.DMA,
        pltpu.SemaphoreType.DMA,
        pltpu.SemaphoreType.DMA,
    ],
)


def _tc_body(part_ref, nm_ref, out_ref):
    p = jnp.sum(part_ref[...])
    nm = nm_ref[...]
    nz = (nm != 0.0).astype(jnp.int32)
    a_cnt = jnp.sum(nz)
    lin = lax.iota(jnp.int32, K_MASK)
    hit = jnp.sum(jnp.where((lin == p) & (nz == 1), 1, 0))
    in_topk = p < K_MASK
    inter = jnp.where(in_topk, a_cnt - hit, a_cnt)
    union = jnp.where(in_topk, (K_MASK - 1) + hit, K_MASK)
    iou = inter.astype(jnp.float32) / union.astype(jnp.float32)
    val = jnp.log(jnp.full((8, 128), 1.0 / iou, jnp.float32))
    out_ref[0, 0] = val[0, 0]


_tc_final = pl.pallas_call(
    _tc_body,
    out_shape=jax.ShapeDtypeStruct((1, 1), jnp.float32),
    in_specs=[
        pl.BlockSpec(memory_space=pltpu.VMEM),
        pl.BlockSpec(memory_space=pltpu.VMEM),
    ],
    out_specs=pl.BlockSpec(memory_space=pltpu.SMEM),
)


def kernel(net_mask, depth_hat, depth, k):
    partials = _sc_count(depth_hat, depth)
    out = _tc_final(partials, net_mask)
    return out[0, 0]


# speedup vs baseline: 67.2469x; 1.0287x over previous
import jax
import jax.numpy as jnp
from jax import lax
from jax.experimental import pallas as pl
from jax.experimental.pallas import tpu as pltpu
from jax.experimental.pallas import tpu_sc as plsc

N_TOTAL = 2097152
K_MASK = 65536
NC = 2
NS = 16
LANES = 16
NW = NC * NS
PER_W = N_TOTAL // NW
CHUNK = 16384
N_CHUNKS = PER_W // CHUNK
UNROLL = 16
INF_BITS = 0x7F800000


def _sc_body(dh_hbm, d_hbm, out_hbm,
             dh_a, d_a, dh_b, d_b, head_a, head_b, row_v,
             sem_dh_a, sem_d_a, sem_dh_b, sem_d_b):
    wid = lax.axis_index("s") * NC + lax.axis_index("c")
    base = wid * PER_W

    pltpu.sync_copy(dh_hbm.at[pl.ds(0, LANES)], head_a)
    pltpu.sync_copy(d_hbm.at[pl.ds(0, LANES)], head_b)
    diff0 = jnp.abs(head_a[...] - head_b[...])
    g_head = jnp.exp(-(diff0 * diff0))
    head_a[...] = g_head
    g0v = plsc.load_gather(head_a, [jnp.zeros((LANES,), jnp.int32)])

    def bis(_, lh):
        lo, hi = lh
        mid = lo + lax.div(hi - lo, 2)
        t = plsc.bitcast(mid, jnp.float32)
        pfalse = jnp.exp(-t) <= g0v
        return (jnp.where(pfalse, lo, mid + 1),
                jnp.where(pfalse, mid, hi))

    _, hi = lax.fori_loop(
        0, 32, bis,
        (jnp.zeros((LANES,), jnp.int32),
         jnp.full((LANES,), INF_BITS, jnp.int32)))
    tstar = plsc.bitcast(hi, jnp.float32)

    dh_bufs, d_bufs = (dh_a, dh_b), (d_a, d_b)
    dh_sems, d_sems = (sem_dh_a, sem_dh_b), (sem_d_a, sem_d_b)

    def start(c, b):
        off = base + c * CHUNK
        return (pltpu.async_copy(dh_hbm.at[pl.ds(off, CHUNK)],
                                 dh_bufs[b], dh_sems[b]),
                pltpu.async_copy(d_hbm.at[pl.ds(off, CHUNK)],
                                 d_bufs[b], d_sems[b]))

    acc = jnp.zeros((LANES,), jnp.int32)
    pend = start(0, 0)
    for c in range(N_CHUNKS):
        nxt = start(c + 1, (c + 1) % 2) if c + 1 < N_CHUNKS else None
        for h in pend:
            h.wait()
        dh_v, d_v = dh_bufs[c % 2], d_bufs[c % 2]

        def inner(i, a, dh_v=dh_v, d_v=d_v):
            s0 = i * (LANES * UNROLL)
            for u in range(UNROLL):
                s = s0 + u * LANES
                x = dh_v[pl.ds(s, LANES)] - d_v[pl.ds(s, LANES)]
                a = a + jnp.where(x * x < tstar, 1, 0).astype(jnp.int32)
            return a

        acc = lax.fori_loop(0, CHUNK // (LANES * UNROLL), inner, acc)
        pend = nxt
    row_v[...] = acc
    pltpu.sync_copy(row_v, out_hbm.at[wid])


_sc_count = pl.kernel(
    _sc_body,
    out_type=jax.ShapeDtypeStruct((NW, LANES), jnp.int32),
    mesh=plsc.VectorSubcoreMesh(
        core_axis_name="c", subcore_axis_name="s",
        num_cores=NC, num_subcores=NS),
    compiler_params=pltpu.CompilerParams(
        needs_layout_passes=False,
        disable_bounds_checks=True,
        disable_semaphore_checks=True,
        skip_device_barrier=True,
    ),
    scratch_types=[
        pltpu.VMEM((CHUNK,), jnp.float32),
        pltpu.VMEM((CHUNK,), jnp.float32),
        pltpu.VMEM((CHUNK,), jnp.float32),
        pltpu.VMEM((CHUNK,), jnp.float32),
        pltpu.VMEM((LANES,), jnp.float32),
        pltpu.VMEM((LANES,), jnp.float32),
        pltpu.VMEM((LANES,), jnp.int32),
        pltpu.SemaphoreType.DMA,
        pltpu.SemaphoreType.DMA,
        pltpu.SemaphoreType.DMA,
        pltpu.SemaphoreType.DMA,
    ],
)


def _tc_body(part_ref, nm_ref, out_ref):
    p = jnp.sum(part_ref[...])
    nm = nm_ref[...]
    nz = (nm != 0.0).astype(jnp.int32)
    a_cnt = jnp.sum(nz)
    rows = lax.broadcasted_iota(jnp.int32, nm.shape, 0)
    cols = lax.broadcasted_iota(jnp.int32, nm.shape, 1)
    lin = rows * nm.shape[1] + cols
    hit = jnp.sum(jnp.where((lin == p) & (nz == 1), 1, 0))
    in_topk = p < K_MASK
    inter = jnp.where(in_topk, a_cnt - hit, a_cnt)
    union = jnp.where(in_topk, (K_MASK - 1) + hit, K_MASK)
    iou = inter.astype(jnp.float32) / union.astype(jnp.float32)
    val = jnp.log(jnp.full((8, 128), 1.0 / iou, jnp.float32))
    out_ref[0, 0] = val[0, 0]


_tc_final = pl.pallas_call(
    _tc_body,
    out_shape=jax.ShapeDtypeStruct((1, 1), jnp.float32),
    in_specs=[
        pl.BlockSpec(memory_space=pltpu.VMEM),
        pl.BlockSpec(memory_space=pltpu.VMEM),
    ],
    out_specs=pl.BlockSpec(memory_space=pltpu.SMEM),
)


def kernel(net_mask, depth_hat, depth, k):
    partials = _sc_count(depth_hat, depth)
    nm2 = net_mask.reshape(K_MASK // 1024, 1024)
    out = _tc_final(partials, nm2)
    return out[0, 0]


# speedup vs baseline: 71.9712x; 1.0703x over previous
import jax
import jax.numpy as jnp
from jax import lax
from jax.experimental import pallas as pl
from jax.experimental.pallas import tpu as pltpu
from jax.experimental.pallas import tpu_sc as plsc

N_TOTAL = 2097152
K_MASK = 65536
NC = 2
NS = 16
LANES = 16
NW = NC * NS
SPLIT = N_TOTAL // 2
PER_W = (N_TOTAL - SPLIT) // NW
CHUNK = 16384
N_CHUNKS = PER_W // CHUNK
UNROLL = 16
INF_BITS = 0x7F800000
TCA_BLOCK = 131072
TCA_GRID = SPLIT // TCA_BLOCK


def _sc_body(dh_hbm, d_hbm, out_hbm,
             dh_a, d_a, dh_b, d_b, head_a, head_b, row_v,
             sem_dh_a, sem_d_a, sem_dh_b, sem_d_b):
    wid = lax.axis_index("s") * NC + lax.axis_index("c")
    base = SPLIT + wid * PER_W

    pltpu.sync_copy(dh_hbm.at[pl.ds(0, LANES)], head_a)
    pltpu.sync_copy(d_hbm.at[pl.ds(0, LANES)], head_b)
    diff0 = jnp.abs(head_a[...] - head_b[...])
    g_head = jnp.exp(-(diff0 * diff0))
    head_a[...] = g_head
    g0v = plsc.load_gather(head_a, [jnp.zeros((LANES,), jnp.int32)])

    def bis(_, lh):
        lo, hi = lh
        mid = lo + lax.div(hi - lo, 2)
        t = plsc.bitcast(mid, jnp.float32)
        pfalse = jnp.exp(-t) <= g0v
        return (jnp.where(pfalse, lo, mid + 1),
                jnp.where(pfalse, mid, hi))

    _, hi = lax.fori_loop(
        0, 32, bis,
        (jnp.zeros((LANES,), jnp.int32),
         jnp.full((LANES,), INF_BITS, jnp.int32)))
    tstar = plsc.bitcast(hi, jnp.float32)

    dh_bufs, d_bufs = (dh_a, dh_b), (d_a, d_b)
    dh_sems, d_sems = (sem_dh_a, sem_dh_b), (sem_d_a, sem_d_b)

    def start(c, b):
        off = base + c * CHUNK
        return (pltpu.async_copy(dh_hbm.at[pl.ds(off, CHUNK)],
                                 dh_bufs[b], dh_sems[b]),
                pltpu.async_copy(d_hbm.at[pl.ds(off, CHUNK)],
                                 d_bufs[b], d_sems[b]))

    acc = jnp.zeros((LANES,), jnp.int32)
    pend = start(0, 0)
    for c in range(N_CHUNKS):
        nxt = start(c + 1, (c + 1) % 2) if c + 1 < N_CHUNKS else None
        for h in pend:
            h.wait()
        dh_v, d_v = dh_bufs[c % 2], d_bufs[c % 2]

        def inner(i, a, dh_v=dh_v, d_v=d_v):
            s0 = i * (LANES * UNROLL)
            for u in range(UNROLL):
                s = s0 + u * LANES
                x = dh_v[pl.ds(s, LANES)] - d_v[pl.ds(s, LANES)]
                a = a + jnp.where(x * x < tstar, 1, 0).astype(jnp.int32)
            return a

        acc = lax.fori_loop(0, CHUNK // (LANES * UNROLL), inner, acc)
        pend = nxt
    row_v[...] = acc
    pltpu.sync_copy(row_v, out_hbm.at[wid])


_sc_count = pl.kernel(
    _sc_body,
    out_type=jax.ShapeDtypeStruct((NW, LANES), jnp.int32),
    mesh=plsc.VectorSubcoreMesh(
        core_axis_name="c", subcore_axis_name="s",
        num_cores=NC, num_subcores=NS),
    compiler_params=pltpu.CompilerParams(
        needs_layout_passes=False,
        disable_bounds_checks=True,
        disable_semaphore_checks=True,
        skip_device_barrier=True,
    ),
    scratch_types=[
        pltpu.VMEM((CHUNK,), jnp.float32),
        pltpu.VMEM((CHUNK,), jnp.float32),
        pltpu.VMEM((CHUNK,), jnp.float32),
        pltpu.VMEM((CHUNK,), jnp.float32),
        pltpu.VMEM((LANES,), jnp.float32),
        pltpu.VMEM((LANES,), jnp.float32),
        pltpu.VMEM((LANES,), jnp.int32),
        pltpu.SemaphoreType.DMA,
        pltpu.SemaphoreType.DMA,
        pltpu.SemaphoreType.DMA,
        pltpu.SemaphoreType.DMA,
    ],
)


def _tca_body(dh_ref, d_ref, out_ref, g0_smem):
    i = pl.program_id(0)

    @pl.when(i == 0)
    def _():
        head = dh_ref[pl.ds(0, 128)] - d_ref[pl.ds(0, 128)]
        ahead = jnp.abs(head)
        ghead = jnp.exp(-(ahead * ahead))
        lane = lax.iota(jnp.int32, 128)
        g0_smem[0] = jnp.sum(jnp.where(lane == 0, ghead, 0.0))
        out_ref[0, 0] = 0

    g0 = g0_smem[0]
    x = dh_ref[...] - d_ref[...]
    ax = jnp.abs(x)
    g = jnp.exp(-(ax * ax))
    cnt = jnp.sum(jnp.where(g > g0, 1, 0).astype(jnp.int32))
    out_ref[0, 0] += cnt


_tc_count = pl.pallas_call(
    _tca_body,
    grid=(TCA_GRID,),
    out_shape=jax.ShapeDtypeStruct((1, 1), jnp.int32),
    in_specs=[
        pl.BlockSpec((TCA_BLOCK,), lambda i: (i,)),
        pl.BlockSpec((TCA_BLOCK,), lambda i: (i,)),
    ],
    out_specs=pl.BlockSpec(memory_space=pltpu.SMEM),
    scratch_shapes=[pltpu.SMEM((1,), jnp.float32)],
)


def _tc_body(part_ref, tca_ref, nm_ref, out_ref):
    p = jnp.sum(part_ref[...]) + tca_ref[0, 0]
    nm = nm_ref[...]
    nz = (nm != 0.0).astype(jnp.int32)
    a_cnt = jnp.sum(nz)
    rows = lax.broadcasted_iota(jnp.int32, nm.shape, 0)
    cols = lax.broadcasted_iota(jnp.int32, nm.shape, 1)
    lin = rows * nm.shape[1] + cols
    hit = jnp.sum(jnp.where((lin == p) & (nz == 1), 1, 0))
    in_topk = p < K_MASK
    inter = jnp.where(in_topk, a_cnt - hit, a_cnt)
    union = jnp.where(in_topk, (K_MASK - 1) + hit, K_MASK)
    iou = inter.astype(jnp.float32) / union.astype(jnp.float32)
    val = jnp.log(jnp.full((8, 128), 1.0 / iou, jnp.float32))
    out_ref[0, 0] = val[0, 0]


_tc_final = pl.pallas_call(
    _tc_body,
    out_shape=jax.ShapeDtypeStruct((1, 1), jnp.float32),
    in_specs=[
        pl.BlockSpec(memory_space=pltpu.VMEM),
        pl.BlockSpec(memory_space=pltpu.SMEM),
        pl.BlockSpec(memory_space=pltpu.VMEM),
    ],
    out_specs=pl.BlockSpec(memory_space=pltpu.SMEM),
)


def kernel(net_mask, depth_hat, depth, k):
    partials = _sc_count(depth_hat, depth)
    tca = _tc_count(depth_hat, depth)
    nm2 = net_mask.reshape(K_MASK // 1024, 1024)
    out = _tc_final(partials, tca, nm2)
    return out[0, 0]


# speedup vs baseline: 72.9529x; 1.0136x over previous
import jax
import jax.numpy as jnp
from jax import lax
from jax.experimental import pallas as pl
from jax.experimental.pallas import tpu as pltpu
from jax.experimental.pallas import tpu_sc as plsc

N_TOTAL = 2097152
K_MASK = 65536
NC = 2
NS = 16
LANES = 16
NW = NC * NS
SPLIT = N_TOTAL // 2
PER_W = (N_TOTAL - SPLIT) // NW
CHUNK = 16384
N_CHUNKS = PER_W // CHUNK
UNROLL = 16
INF_BITS = 0x7F800000
TCA_BLOCK = 131072
TCA_GRID = SPLIT // TCA_BLOCK


def _sc_body(dh_hbm, d_hbm, out_hbm,
             dh_a, d_a, dh_b, d_b, head_a, head_b, row_v,
             sem_dh_a, sem_d_a, sem_dh_b, sem_d_b):
    wid = lax.axis_index("s") * NC + lax.axis_index("c")
    base = SPLIT + wid * PER_W

    pltpu.sync_copy(dh_hbm.at[pl.ds(0, LANES)], head_a)
    pltpu.sync_copy(d_hbm.at[pl.ds(0, LANES)], head_b)
    diff0 = jnp.abs(head_a[...] - head_b[...])
    g_head = jnp.exp(-(diff0 * diff0))
    head_a[...] = g_head
    g0v = plsc.load_gather(head_a, [jnp.zeros((LANES,), jnp.int32)])

    def bis(_, lh):
        lo, hi = lh
        mid = lo + lax.div(hi - lo, 2)
        t = plsc.bitcast(mid, jnp.float32)
        pfalse = jnp.exp(-t) <= g0v
        return (jnp.where(pfalse, lo, mid + 1),
                jnp.where(pfalse, mid, hi))

    _, hi = lax.fori_loop(
        0, 32, bis,
        (jnp.zeros((LANES,), jnp.int32),
         jnp.full((LANES,), INF_BITS, jnp.int32)))
    tstar = plsc.bitcast(hi, jnp.float32)

    dh_bufs, d_bufs = (dh_a, dh_b), (d_a, d_b)
    dh_sems, d_sems = (sem_dh_a, sem_dh_b), (sem_d_a, sem_d_b)

    def start(c, b):
        off = base + c * CHUNK
        return (pltpu.async_copy(dh_hbm.at[pl.ds(off, CHUNK)],
                                 dh_bufs[b], dh_sems[b]),
                pltpu.async_copy(d_hbm.at[pl.ds(off, CHUNK)],
                                 d_bufs[b], d_sems[b]))

    acc = jnp.zeros((LANES,), jnp.int32)
    pend = start(0, 0)
    for c in range(N_CHUNKS):
        nxt = start(c + 1, (c + 1) % 2) if c + 1 < N_CHUNKS else None
        for h in pend:
            h.wait()
        dh_v, d_v = dh_bufs[c % 2], d_bufs[c % 2]

        def inner(i, a, dh_v=dh_v, d_v=d_v):
            s0 = i * (LANES * UNROLL)
            for u in range(UNROLL):
                s = s0 + u * LANES
                x = dh_v[pl.ds(s, LANES)] - d_v[pl.ds(s, LANES)]
                a = a + jnp.where(x * x < tstar, 1, 0).astype(jnp.int32)
            return a

        acc = lax.fori_loop(0, CHUNK // (LANES * UNROLL), inner, acc)
        pend = nxt
    row_v[...] = acc
    pltpu.sync_copy(row_v, out_hbm.at[wid])


_sc_count = pl.kernel(
    _sc_body,
    out_type=jax.ShapeDtypeStruct((NW, LANES), jnp.int32),
    mesh=plsc.VectorSubcoreMesh(
        core_axis_name="c", subcore_axis_name="s",
        num_cores=NC, num_subcores=NS),
    compiler_params=pltpu.CompilerParams(
        needs_layout_passes=False,
        disable_bounds_checks=True,
        disable_semaphore_checks=True,
        skip_device_barrier=True,
    ),
    scratch_types=[
        pltpu.VMEM((CHUNK,), jnp.float32),
        pltpu.VMEM((CHUNK,), jnp.float32),
        pltpu.VMEM((CHUNK,), jnp.float32),
        pltpu.VMEM((CHUNK,), jnp.float32),
        pltpu.VMEM((LANES,), jnp.float32),
        pltpu.VMEM((LANES,), jnp.float32),
        pltpu.VMEM((LANES,), jnp.int32),
        pltpu.SemaphoreType.DMA,
        pltpu.SemaphoreType.DMA,
        pltpu.SemaphoreType.DMA,
        pltpu.SemaphoreType.DMA,
    ],
)


def _tca_body(dh_ref, d_ref, nm_ref, out_ref, g0_smem):
    i = pl.program_id(0)

    @pl.when(i == 0)
    def _():
        head = (dh_ref[pl.ds(0, 128)] - d_ref[pl.ds(0, 128)]).reshape(1, 128)
        ahead = jnp.abs(head)
        ghead = jnp.exp(-(ahead * ahead))
        lane = lax.broadcasted_iota(jnp.int32, (1, 128), 1)
        g0_smem[0] = jnp.sum(jnp.where(lane == 0, ghead, 0.0))
        out_ref[0, 0] = 0
        out_ref[0, 1] = jnp.sum((nm_ref[...] != 0.0).astype(jnp.int32))

    g0 = g0_smem[0]
    x = (dh_ref[...] - d_ref[...]).reshape(TCA_BLOCK // 128, 128)
    ax = jnp.abs(x)
    g = jnp.exp(-(ax * ax))
    cnt = jnp.sum(jnp.where(g > g0, 1, 0).astype(jnp.int32))
    out_ref[0, 0] += cnt


_tc_count = pl.pallas_call(
    _tca_body,
    grid=(TCA_GRID,),
    out_shape=jax.ShapeDtypeStruct((1, 2), jnp.int32),
    in_specs=[
        pl.BlockSpec((TCA_BLOCK,), lambda i: (i,)),
        pl.BlockSpec((TCA_BLOCK,), lambda i: (i,)),
        pl.BlockSpec((K_MASK // 1024, 1024), lambda i: (0, 0)),
    ],
    out_specs=pl.BlockSpec(memory_space=pltpu.SMEM),
    scratch_shapes=[pltpu.SMEM((1,), jnp.float32)],
)


def _tc_body(part_ref, tca_ref, nm_ref, out_ref):
    p = jnp.sum(part_ref[...]) + tca_ref[0, 0]
    a_cnt = tca_ref[0, 1]
    nm = nm_ref[...]
    rows = lax.broadcasted_iota(jnp.int32, nm.shape, 0)
    cols = lax.broadcasted_iota(jnp.int32, nm.shape, 1)
    lin = rows * nm.shape[1] + cols
    hit = jnp.sum(jnp.where((lin == p) & (nm != 0.0), 1, 0))
    in_topk = p < K_MASK
    inter = jnp.where(in_topk, a_cnt - hit, a_cnt)
    union = jnp.where(in_topk, (K_MASK - 1) + hit, K_MASK)
    iou = inter.astype(jnp.float32) / union.astype(jnp.float32)
    val = jnp.log(jnp.full((8, 128), 1.0 / iou, jnp.float32))
    out_ref[0, 0] = val[0, 0]


_tc_final = pl.pallas_call(
    _tc_body,
    out_shape=jax.ShapeDtypeStruct((1, 1), jnp.float32),
    in_specs=[
        pl.BlockSpec(memory_space=pltpu.VMEM),
        pl.BlockSpec(memory_space=pltpu.SMEM),
        pl.BlockSpec(memory_space=pltpu.VMEM),
    ],
    out_specs=pl.BlockSpec(memory_space=pltpu.SMEM),
)


def kernel(net_mask, depth_hat, depth, k):
    partials = _sc_count(depth_hat, depth)
    nm2 = net_mask.reshape(K_MASK // 1024, 1024)
    tca = _tc_count(depth_hat, depth, nm2)
    out = _tc_final(partials, tca, nm2)
    return out[0, 0]
